# baseline (speedup 1.0000x reference)
import functools

import jax
import jax.numpy as jnp
from jax import lax
from jax.experimental import pallas as pl
from jax.experimental.pallas import tpu as pltpu
from jax.experimental.pallas import tpu_sc as plsc

BATCH = 16384
NC = 2
NS = 16
NW = NC * NS
C = BATCH // NW
E = 2 * C
CHUNK = 128
NCHUNK = E // CHUNK


def _body(x1_hbm, y1_hbm, group_hbm, ic_hbm, ex_hbm, ey_hbm, tab_hbm,
          out_hbm, g_v, eidx_v, rows_v, x_v, y_v, ic_v, ex_v, ey_v, sem):
    wid = lax.axis_index("s") * NC + lax.axis_index("c")
    base = wid * C

    pltpu.sync_copy(group_hbm.at[pl.ds(base, C)], g_v)
    pltpu.sync_copy(x1_hbm.at[pl.ds(base, C)], x_v)
    pltpu.sync_copy(y1_hbm.at[pl.ds(base, C)], y_v)
    pltpu.sync_copy(ic_hbm, ic_v.at[pl.ds(0, 2)])
    pltpu.sync_copy(ex_hbm, ex_v.at[pl.ds(0, 2)])
    pltpu.sync_copy(ey_hbm, ey_v.at[pl.ds(0, 2)])

    lane = lax.iota(jnp.int32, 16)
    half = lane >> 1
    cidx = lane & 1

    copies = []
    for j in range(NCHUNK):
        def prep(t, carry, j=j):
            gd = plsc.load_gather(g_v, [j * 64 + t * 8 + half])
            eidx_v[j, pl.ds(t * 16, 16)] = gd * 2 + cidx
            return carry

        lax.fori_loop(0, 8, prep, 0)
        copies.append(pltpu.async_copy(
            tab_hbm.at[eidx_v.at[j]],
            rows_v.at[pl.ds(j * CHUNK, CHUNK)], sem))

    for c in copies:
        c.wait()

    icv = plsc.load_gather(ic_v, [cidx])
    exv = plsc.load_gather(ex_v, [cidx])
    eyv = plsc.load_gather(ey_v, [cidx])

    def step(m, carry):
        xd = plsc.load_gather(x_v, [m * 8 + half])
        yd = plsc.load_gather(y_v, [m * 8 + half])
        rv = rows_v[pl.ds(m * 16, 16)]
        rows_v[pl.ds(m * 16, 16)] = rv + icv + exv * xd + eyv * yd
        return carry

    lax.fori_loop(0, E // 16, step, 0)

    pltpu.sync_copy(rows_v, out_hbm.at[pl.ds(wid * E, E)])


@jax.jit
def _mlmm(x1, y1, group, intercept, effect_x1, effect_y1, group_effects):
    kern = functools.partial(
        pl.kernel,
        out_type=jax.ShapeDtypeStruct((BATCH * 2,), jnp.float32),
        mesh=plsc.VectorSubcoreMesh(core_axis_name="c", subcore_axis_name="s"),
        compiler_params=pltpu.CompilerParams(needs_layout_passes=False,
                                             use_tc_tiling_on_sc=False),
        scratch_types=[
            pltpu.VMEM((C,), jnp.int32),
            pltpu.VMEM((NCHUNK, CHUNK), jnp.int32),
            pltpu.VMEM((E,), jnp.float32),
            pltpu.VMEM((C,), jnp.float32),
            pltpu.VMEM((C,), jnp.float32),
            pltpu.VMEM((16,), jnp.float32),
            pltpu.VMEM((16,), jnp.float32),
            pltpu.VMEM((16,), jnp.float32),
            pltpu.SemaphoreType.DMA,
        ],
    )(_body)
    flat = kern(x1, y1, group, intercept, effect_x1, effect_y1,
                group_effects.reshape(-1))
    return flat.reshape(BATCH, 2)


def kernel(x1, y1, group, intercept, effect_x1, effect_y1, group_effects):
    return _mlmm(x1, y1, group.astype(jnp.int32), intercept, effect_x1,
                 effect_y1, group_effects)


# speedup vs baseline: 24.8320x; 24.8320x over previous
import functools

import jax
import jax.numpy as jnp
from jax import lax
from jax.experimental import pallas as pl
from jax.experimental.pallas import tpu as pltpu
from jax.experimental.pallas import tpu_sc as plsc

BATCH = 16384
NGROUP = 1000000
TILE = 128
NTILE_FULL = NGROUP // TILE
MAIN = NTILE_FULL * TILE
NTAIL = NGROUP - MAIN
NC = 2
NS = 16
NW = NC * NS
C = BATCH // NW
E = 2 * C
NCHUNK = E // TILE


def _body(x1_hbm, y1_hbm, group_hbm, ic_hbm, ex_hbm, ey_hbm, tab_hbm,
          tail_hbm, out_hbm, g_v, eidx_v, rows_v, x_v, y_v, tail_v,
          ic_v, ex_v, ey_v, sem):
    wid = lax.axis_index("s") * NC + lax.axis_index("c")
    base = wid * C

    pltpu.sync_copy(group_hbm.at[pl.ds(base, C)], g_v)

    for j in range(NCHUNK):
        tt, c = j >> 1, j & 1

        def prep(t, carry, tt=tt, c=c, j=j):
            g16 = g_v[pl.ds(tt * TILE + t * 16, 16)]
            gc = jnp.minimum(g16, MAIN - 1)
            e = ((gc >> 7) << 8) + (c << 7) + (gc & 127)
            eidx_v[j, pl.ds(t * 16, 16)] = e
            return carry

        lax.fori_loop(0, 8, prep, 0)

    copies = []
    for j in range(NCHUNK):
        copies.append(pltpu.async_copy(
            tab_hbm.at[eidx_v.at[j]],
            rows_v.at[pl.ds(j * TILE, TILE)], sem))

    pltpu.sync_copy(x1_hbm.at[pl.ds(base, C)], x_v)
    pltpu.sync_copy(y1_hbm.at[pl.ds(base, C)], y_v)
    pltpu.sync_copy(tail_hbm, tail_v)
    pltpu.sync_copy(ic_hbm, ic_v.at[pl.ds(0, 2)])
    pltpu.sync_copy(ex_hbm, ex_v.at[pl.ds(0, 2)])
    pltpu.sync_copy(ey_hbm, ey_v.at[pl.ds(0, 2)])

    for cp in copies:
        cp.wait()

    lane = lax.iota(jnp.int32, 16)
    zero16 = lane * 0
    coef = []
    for c in range(2):
        cvec = zero16 + c
        coef.append((plsc.load_gather(ic_v, [cvec]),
                     plsc.load_gather(ex_v, [cvec]),
                     plsc.load_gather(ey_v, [cvec])))

    for j in range(NCHUNK):
        tt, c = j >> 1, j & 1
        icc, exc, eyc = coef[c]

        def fma(t, carry, tt=tt, c=c, j=j, icc=icc, exc=exc, eyc=eyc):
            r = tt * TILE + t * 16
            g16 = g_v[pl.ds(r, 16)]
            x16 = x_v[pl.ds(r, 16)]
            y16 = y_v[pl.ds(r, 16)]
            rv = rows_v[pl.ds(j * TILE + t * 16, 16)]
            tidx = jnp.maximum(g16 - MAIN, 0) * 2 + c
            tv = plsc.load_gather(tail_v, [tidx])
            val = jnp.where(g16 >= MAIN, tv, rv)
            rows_v[pl.ds(j * TILE + t * 16, 16)] = (
                val + icc + exc * x16 + eyc * y16)
            return carry

        lax.fori_loop(0, 8, fma, 0)

    pltpu.sync_copy(rows_v, out_hbm.at[pl.ds(wid * E, E)])


@jax.jit
def _mlmm(x1, y1, group, intercept, effect_x1, effect_y1, group_effects):
    tab = (group_effects[:MAIN]
           .reshape(NTILE_FULL, TILE, 2)
           .transpose(0, 2, 1)
           .reshape(-1))
    tail = group_effects[MAIN:].reshape(-1)
    kern = functools.partial(
        pl.kernel,
        out_type=jax.ShapeDtypeStruct((BATCH * 2,), jnp.float32),
        mesh=plsc.VectorSubcoreMesh(core_axis_name="c", subcore_axis_name="s"),
        compiler_params=pltpu.CompilerParams(needs_layout_passes=False,
                                             use_tc_tiling_on_sc=False),
        scratch_types=[
            pltpu.VMEM((C,), jnp.int32),
            pltpu.VMEM((NCHUNK, TILE), jnp.int32),
            pltpu.VMEM((E,), jnp.float32),
            pltpu.VMEM((C,), jnp.float32),
            pltpu.VMEM((C,), jnp.float32),
            pltpu.VMEM((2 * NTAIL,), jnp.float32),
            pltpu.VMEM((16,), jnp.float32),
            pltpu.VMEM((16,), jnp.float32),
            pltpu.VMEM((16,), jnp.float32),
            pltpu.SemaphoreType.DMA,
        ],
    )(_body)
    flat = kern(x1, y1, group, intercept, effect_x1, effect_y1, tab, tail)
    return (flat.reshape(BATCH // TILE, 2, TILE)
            .transpose(0, 2, 1)
            .reshape(BATCH, 2))


def kernel(x1, y1, group, intercept, effect_x1, effect_y1, group_effects):
    return _mlmm(x1, y1, group.astype(jnp.int32), intercept, effect_x1,
                 effect_y1, group_effects)


# speedup vs baseline: 24.9076x; 1.0030x over previous
import functools

import jax
import jax.numpy as jnp
from jax import lax
from jax.experimental import pallas as pl
from jax.experimental.pallas import tpu as pltpu
from jax.experimental.pallas import tpu_sc as plsc

BATCH = 16384
NGROUP = 1000000
TILE = 128
NTILE_FULL = NGROUP // TILE
MAIN = NTILE_FULL * TILE
NTAIL = NGROUP - MAIN
NC = 2
NS = 16
NW = NC * NS
C = BATCH // NW
E = 2 * C
NCHUNK = E // TILE


def _body(x1_hbm, y1_hbm, group_hbm, ic_hbm, ex_hbm, ey_hbm, tab_hbm,
          tail_hbm, out_hbm, g_v, eidx_v, rows_v, x_v, y_v, tail_v,
          ic_v, ex_v, ey_v, sem):
    wid = lax.axis_index("s") * NC + lax.axis_index("c")
    base = wid * C

    pltpu.sync_copy(group_hbm.at[pl.ds(base, C)], g_v)

    for j in range(NCHUNK):
        tt, c = j >> 1, j & 1

        def prep(t, carry, tt=tt, c=c, j=j):
            g16 = g_v[pl.ds(tt * TILE + t * 16, 16)]
            gc = jnp.minimum(g16, MAIN - 1)
            e = ((gc >> 7) << 8) + (c << 7) + (gc & 127)
            eidx_v[j, pl.ds(t * 16, 16)] = e
            return carry

        lax.fori_loop(0, 8, prep, 0)

    copies = []
    for j in range(NCHUNK):
        copies.append(pltpu.async_copy(
            tab_hbm.at[eidx_v.at[j]],
            rows_v.at[pl.ds(j * TILE, TILE)], sem))

    pltpu.sync_copy(x1_hbm.at[pl.ds(base, C)], x_v)
    pltpu.sync_copy(y1_hbm.at[pl.ds(base, C)], y_v)
    pltpu.sync_copy(tail_hbm, tail_v)
    pltpu.sync_copy(ic_hbm, ic_v.at[pl.ds(0, 2)])
    pltpu.sync_copy(ex_hbm, ex_v.at[pl.ds(0, 2)])
    pltpu.sync_copy(ey_hbm, ey_v.at[pl.ds(0, 2)])

    for cp in copies:
        cp.wait()

    zeros16 = jnp.zeros((16,), jnp.float32)
    icl, exl, eyl = ic_v[...], ex_v[...], ey_v[...]
    coef = []
    for c in range(2):
        coef.append((zeros16 + icl[c], zeros16 + exl[c],
                     zeros16 + eyl[c]))

    for j in range(NCHUNK):
        tt, c = j >> 1, j & 1
        icc, exc, eyc = coef[c]

        def fma(t, carry, tt=tt, c=c, j=j, icc=icc, exc=exc, eyc=eyc):
            r = tt * TILE + t * 16
            g16 = g_v[pl.ds(r, 16)]
            x16 = x_v[pl.ds(r, 16)]
            y16 = y_v[pl.ds(r, 16)]
            rv = rows_v[pl.ds(j * TILE + t * 16, 16)]
            tidx = jnp.maximum(g16 - MAIN, 0) * 2 + c
            tv = plsc.load_gather(tail_v, [tidx])
            val = jnp.where(g16 >= MAIN, tv, rv)
            rows_v[pl.ds(j * TILE + t * 16, 16)] = (
                val + icc + exc * x16 + eyc * y16)
            return carry

        lax.fori_loop(0, 8, fma, 0)

    pltpu.sync_copy(rows_v, out_hbm.at[pl.ds(wid * E, E)])


@jax.jit
def _mlmm(x1, y1, group, intercept, effect_x1, effect_y1, group_effects):
    tab = (group_effects[:MAIN]
           .reshape(NTILE_FULL, TILE, 2)
           .transpose(0, 2, 1)
           .reshape(-1))
    tail = group_effects[MAIN:].reshape(-1)
    kern = functools.partial(
        pl.kernel,
        out_type=jax.ShapeDtypeStruct((BATCH * 2,), jnp.float32),
        mesh=plsc.VectorSubcoreMesh(core_axis_name="c", subcore_axis_name="s"),
        compiler_params=pltpu.CompilerParams(needs_layout_passes=False,
                                             use_tc_tiling_on_sc=False),
        scratch_types=[
            pltpu.VMEM((C,), jnp.int32),
            pltpu.VMEM((NCHUNK, TILE), jnp.int32),
            pltpu.VMEM((E,), jnp.float32),
            pltpu.VMEM((C,), jnp.float32),
            pltpu.VMEM((C,), jnp.float32),
            pltpu.VMEM((2 * NTAIL,), jnp.float32),
            pltpu.VMEM((16,), jnp.float32),
            pltpu.VMEM((16,), jnp.float32),
            pltpu.VMEM((16,), jnp.float32),
            pltpu.SemaphoreType.DMA,
        ],
    )(_body)
    flat = kern(x1, y1, group, intercept, effect_x1, effect_y1, tab, tail)
    return (flat.reshape(BATCH // TILE, 2, TILE)
            .transpose(0, 2, 1)
            .reshape(BATCH, 2))


def kernel(x1, y1, group, intercept, effect_x1, effect_y1, group_effects):
    return _mlmm(x1, y1, group.astype(jnp.int32), intercept, effect_x1,
                 effect_y1, group_effects)
